# Initial kernel scaffold; baseline (speedup 1.0000x reference)
import jax
import jax.numpy as jnp
from jax.experimental import pallas as pl


def kernel(x, params, edge_index, batch):
    raise NotImplementedError("write your pallas kernel here")


# speedup vs baseline: 1.0257x; 1.0257x over previous
import functools

import jax
import jax.numpy as jnp
from jax import lax
from jax.experimental import pallas as pl
from jax.experimental.pallas import tpu as pltpu
from jax.experimental.pallas import tpu_sc as plsc


def _chunk_size(e_per_w):
    for c in range(128, 7, -1):
        if c % 8 == 0 and e_per_w % c == 0:
            return c
    return None


def _sc_row_gather(x, idx):
    n_idx = idx.shape[0]
    d0 = x.shape[1]
    d = ((d0 + 127) // 128) * 128
    if d != d0:
        x = jnp.pad(x, ((0, 0), (0, d - d0)))
    info = plsc.get_sparse_core_info()
    ncores, nsub = info.num_cores, info.num_subcores
    nw = ncores * nsub
    n_pad = ((n_idx + 8 * nw - 1) // (8 * nw)) * (8 * nw)
    if n_pad != n_idx:
        idx = jnp.concatenate(
            [idx, jnp.zeros((n_pad - n_idx,), jnp.int32)])
    i_per_w = n_pad // nw
    c = _chunk_size(i_per_w)
    n_chunks = i_per_w // c

    mesh = plsc.VectorSubcoreMesh(core_axis_name="c", subcore_axis_name="s")

    @functools.partial(
        pl.kernel,
        mesh=mesh,
        out_type=jax.ShapeDtypeStruct((n_pad, d), jnp.float32),
        scratch_types=[
            pltpu.VMEM((c,), jnp.int32),
            pltpu.VMEM((c, d), jnp.float32),
            pltpu.SemaphoreType.DMA,
        ],
    )
    def k(x_hbm, idx_hbm, out_hbm, idx_v, rows_v, sem):
        cid = lax.axis_index("c")
        sid = lax.axis_index("s")
        wid = sid * ncores + cid
        base = wid * i_per_w

        def body(i, carry):
            off = base + i * c
            pltpu.sync_copy(idx_hbm.at[pl.ds(off, c)], idx_v)
            pltpu.async_copy(x_hbm.at[idx_v], rows_v, sem).wait()
            pltpu.sync_copy(rows_v, out_hbm.at[pl.ds(off, c)])
            return carry

        lax.fori_loop(0, n_chunks, body, 0)

    return k(x, idx.astype(jnp.int32))[:n_idx, :d0]


def _bn(h, g, b):
    m = jnp.mean(h, axis=0)
    v = jnp.mean((h - m) ** 2, axis=0)
    return g * (h - m) / jnp.sqrt(v + 1e-5) + b


def _mlp_body(h, p_leaves):
    w1, b1, g1, be1, w2, b2, g2, be2 = p_leaves
    h = jnp.maximum(_bn(jnp.dot(h, w1, preferred_element_type=jnp.float32)
                        + b1, g1, be1), 0.0)
    h = jnp.maximum(_bn(jnp.dot(h, w2, preferred_element_type=jnp.float32)
                        + b2, g2, be2), 0.0)
    return h


def _mlp_leaves(p):
    return [p['l1']['W'], p['l1']['b'], p['g1'], p['be1'],
            p['l2']['W'], p['l2']['b'], p['g2'], p['be2']]


def _tc_gin_mlp(x, aggr, p):
    n = x.shape[0]
    h_out = p['l2']['W'].shape[1]

    def body(x_ref, a_ref, w1, b1, g1, be1, w2, b2, g2, be2, out_ref):
        h = x_ref[...] + a_ref[...]
        out_ref[...] = _mlp_body(h, [w1[...], b1[...], g1[...], be1[...],
                                     w2[...], b2[...], g2[...], be2[...]])

    return pl.pallas_call(
        body,
        out_shape=jax.ShapeDtypeStruct((n, h_out), jnp.float32),
    )(x, aggr, *_mlp_leaves(p))


def _tc_decoder(x1, x2, x3, p3, p2, w_out, b_out):
    nb = x1.shape[0]
    nc = w_out.shape[1]

    def body(x1_ref, x2_ref, x3_ref,
             a1, a2, a3, a4, a5, a6, a7, a8,
             c1, c2, c3, c4, c5, c6, c7, c8,
             w_ref, b_ref, out_ref):
        xd3 = _mlp_body(x3_ref[...], [a1[...], a2[...], a3[...], a4[...],
                                      a5[...], a6[...], a7[...], a8[...]])
        xd2 = _mlp_body(xd3 + x2_ref[...], [c1[...], c2[...], c3[...],
                                            c4[...], c5[...], c6[...],
                                            c7[...], c8[...]])
        logits = jnp.dot(xd2 + x1_ref[...], w_ref[...],
                         preferred_element_type=jnp.float32) + b_ref[...]
        mx = jnp.max(logits, axis=-1, keepdims=True)
        sh = logits - mx
        out_ref[...] = sh - jnp.log(jnp.sum(jnp.exp(sh), axis=-1,
                                            keepdims=True))

    return pl.pallas_call(
        body,
        out_shape=jax.ShapeDtypeStruct((nb, nc), jnp.float32),
    )(x1, x2, x3, *_mlp_leaves(p3), *_mlp_leaves(p2), w_out, b_out)


def _gin_layer(h, ei, ev, p):
    n = h.shape[0]
    aggr = jax.ops.segment_sum(h[ei[0]] * ev[:, None], ei[1],
                               num_segments=n)
    return _tc_gin_mlp(h, aggr, p)


def _topk_pool(x, ei, ev, w, nper, k, nb):
    score = jnp.tanh(x @ w / jnp.linalg.norm(w))
    idx = lax.top_k(score.reshape(nb, nper), k)[1]
    perm = (idx + jnp.arange(nb)[:, None] * nper).reshape(-1)
    new_x = _sc_row_gather(x, perm) * score[perm][:, None]
    nmap = jnp.full((x.shape[0],), -1, dtype=jnp.int32).at[perm].set(
        jnp.arange(nb * k, dtype=jnp.int32))
    keep = (nmap[ei[0]] >= 0) & (nmap[ei[1]] >= 0) & (ev > 0)
    new_ei = jnp.stack([jnp.where(keep, nmap[ei[0]], 0),
                        jnp.where(keep, nmap[ei[1]], 0)])
    return new_x, new_ei, keep.astype(x.dtype)


def _readout(x, nb, k):
    xr = x.reshape(nb, k, x.shape[1])
    return jnp.concatenate([jnp.max(xr, axis=1), jnp.mean(xr, axis=1)],
                           axis=1)


def kernel(x, params, edge_index, batch):
    n, _ = x.shape
    nb = 8
    ev = jnp.ones((edge_index.shape[1],), x.dtype)

    h = _gin_layer(x, edge_index, ev, params['conv1'])
    h, ei, ev = _topk_pool(h, edge_index, ev, params['w_pool1'],
                           n // nb, 1000, nb)
    x1 = _readout(h, nb, 1000)

    h = _gin_layer(h, ei, ev, params['conv2'])
    h, ei, ev = _topk_pool(h, ei, ev, params['w_pool2'], 1000, 800, nb)
    x2 = _readout(h, nb, 800)

    h = _gin_layer(h, ei, ev, params['conv3'])
    h, ei, ev = _topk_pool(h, ei, ev, params['w_pool3'], 800, 640, nb)
    x3 = _readout(h, nb, 640)

    return _tc_decoder(x1, x2, x3, params['dec3'], params['dec2'],
                       params['dec1']['W'], params['dec1']['b'])
